# Initial kernel scaffold; baseline (speedup 1.0000x reference)
import jax
import jax.numpy as jnp
from jax.experimental import pallas as pl


def kernel(x, edge_index, edge_attr, gcn_W, gcn_b, Ws):
    raise NotImplementedError("write your pallas kernel here")


# speedup vs baseline: 8.8441x; 8.8441x over previous
import functools

import jax
import jax.numpy as jnp
from jax import lax
from jax.experimental import pallas as pl
from jax.experimental.pallas import tpu as pltpu
from jax.experimental.pallas import tpu_sc as plsc

N = 10000
E = 320000
D = 128
NT = 16
NC = 2
CT = 10
CROWS = N // CT

A_BLK = 1000
A_EPW = E // (NC * NT)
B_BLK = 400
B_EPW = E // NT


def _a_body(ea_ref, dst_ref, zdeg_ref, degp_ref, w4_ref, deg_sp, ea_v, dst_v,
            w4b16_v, w4b4_v):
    c = lax.axis_index("c")
    s = lax.axis_index("s")
    w = c * NT + s
    crows = pl.ds(s * CROWS, CROWS)

    @pl.when(s < CT)
    def _():
        pltpu.sync_copy(zdeg_ref.at[crows], deg_sp.at[crows])

    plsc.subcore_barrier()

    l = lax.iota(jnp.int32, 16)
    gpat = (l >> 2) * 16 + (l & 3) + 2
    rpat = l >> 2
    cpat = l & 3

    pltpu.sync_copy(zdeg_ref.at[pl.ds(0, A_BLK)], w4b16_v)

    def blk(b, carry):
        base = pl.multiple_of(w * A_EPW + b * A_BLK, 8)
        pltpu.sync_copy(ea_ref.at[pl.ds(base * 16, A_BLK * 16)], ea_v)
        pltpu.sync_copy(dst_ref.at[pl.ds(base, A_BLK)], dst_v)

        def ext(j, carry2):
            w16 = plsc.load_gather(ea_v, [gpat + j * 64])
            plsc.store_scatter(w4b16_v, [rpat + j * 4, cpat], w16)
            plsc.store_scatter(w4b4_v, [rpat + j * 4, cpat], w16)
            return carry2

        lax.fori_loop(0, A_BLK // 4, ext, 0)
        pltpu.sync_copy(w4b4_v, w4_ref.at[pl.ds(base, A_BLK)])
        pltpu.sync_copy(w4b16_v, deg_sp.at[dst_v], add=True)
        return carry

    lax.fori_loop(0, A_EPW // A_BLK, blk, 0)
    plsc.subcore_barrier()

    @pl.when(s < CT)
    def _():
        pltpu.sync_copy(deg_sp.at[crows], degp_ref.at[c, crows])


def _r_body(degp_ref, dis_ref):
    deg = degp_ref[0, :, 0:4] + degp_ref[1, :, 0:4] + 1.0
    dis_ref[...] = jnp.concatenate(
        [lax.rsqrt(deg), jnp.zeros((deg.shape[0], 12), jnp.float32)], axis=-1)


def _b_body(src_ref, dst_ref, w4_ref, dis_ref, xt_ref, zs_ref, s_ref,
            dis_sp, x_sp, s_sp, src_v, dst_v, w_v, a_v, disg_v, xg_v, msg_v):
    c = lax.axis_index("c")
    s = lax.axis_index("s")
    crows = pl.ds(s * CROWS, CROWS)

    @pl.when(s < CT)
    def _():
        pltpu.sync_copy(dis_ref.at[crows], dis_sp.at[crows])

    l = lax.iota(jnp.int32, 16)
    rpat = l >> 2
    cpat = l & 3

    def quarter(qi, carry):
        q = c * 4 + qi

        @pl.when(s < CT)
        def _():
            pltpu.sync_copy(xt_ref.at[q, crows], x_sp.at[crows])
            pltpu.sync_copy(zs_ref.at[crows], s_sp.at[crows])

        plsc.subcore_barrier()

        def blk(b, carry2):
            base = pl.multiple_of(s * B_EPW + b * B_BLK, 8)
            pltpu.sync_copy(src_ref.at[pl.ds(base, B_BLK)], src_v)
            pltpu.sync_copy(dst_ref.at[pl.ds(base, B_BLK)], dst_v)
            pltpu.sync_copy(w4_ref.at[pl.ds(base * 4, B_BLK * 4)], w_v)
            pltpu.sync_copy(dis_sp.at[src_v], disg_v)
            pltpu.sync_copy(x_sp.at[src_v], xg_v)

            def agrp(j, carry3):
                a16 = (plsc.load_gather(disg_v, [rpat + j * 4, cpat])
                       * w_v[pl.ds(j * 16, 16)])
                a_v[pl.ds(j * 16, 16)] = a16
                return carry3

            lax.fori_loop(0, B_BLK * 4 // 16, agrp, 0)

            def edge(e, carry3):
                xv = xg_v[e, pl.ds(0, 16)]
                for i in range(4):
                    ab = plsc.load_gather(
                        a_v, [jnp.broadcast_to(e * 4 + i, (16,))])
                    msg_v[e, i, pl.ds(0, 16)] = ab * xv
                return carry3

            lax.fori_loop(0, B_BLK, edge, 0)
            pltpu.sync_copy(msg_v, s_sp.at[dst_v], add=True)
            return carry2

        lax.fori_loop(0, B_EPW // B_BLK, blk, 0)
        plsc.subcore_barrier()

        @pl.when(s < CT)
        def _():
            pltpu.sync_copy(s_sp.at[crows], s_ref.at[q, crows])

        return carry

    lax.fori_loop(0, 4, quarter, 0)


def _c_body(x_ref, s_ref, degp_ref, w_ref, b_ref, ws_ref, o_ref):
    deg = degp_ref[0, :, 0:4] + degp_ref[1, :, 0:4] + 1.0
    dis = lax.rsqrt(deg)
    dinv = 1.0 / deg
    xb = x_ref[...]
    acc = jnp.zeros(o_ref.shape, jnp.float32)
    for i in range(4):
        si = jnp.concatenate([s_ref[q, :, i, :] for q in range(8)], axis=-1)
        g = dis[:, i:i + 1] * si + dinv[:, i:i + 1] * xb
        agg = jnp.maximum(jnp.dot(g, w_ref[i], preferred_element_type=jnp.float32)
                          + b_ref[i], 0.0)
        acc = acc + jnp.dot(agg, ws_ref[i], preferred_element_type=jnp.float32)
    o_ref[...] = acc


@jax.jit
def kernel(x, edge_index, edge_attr, gcn_W, gcn_b, Ws):
    f32 = jnp.float32
    mesh = plsc.VectorSubcoreMesh(core_axis_name="c", subcore_axis_name="s")

    phase_a = pl.kernel(
        _a_body,
        out_type=(jax.ShapeDtypeStruct((NC, N, 16), f32),
                  jax.ShapeDtypeStruct((E, 4), f32)),
        mesh=mesh,
        compiler_params=pltpu.CompilerParams(needs_layout_passes=False, use_tc_tiling_on_sc=False),
        scratch_types=[
            pltpu.VMEM_SHARED((N, 16), f32),
            pltpu.VMEM((A_BLK * 16,), f32),
            pltpu.VMEM((A_BLK,), jnp.int32),
            pltpu.VMEM((A_BLK, 16), f32),
            pltpu.VMEM((A_BLK, 4), f32),
        ],
    )
    ea_flat = edge_attr.reshape(E * 16)
    src_h = edge_index[0]
    dst_h = edge_index[1]
    z_deg = jnp.zeros((N, 16), f32)
    degp, w4 = phase_a(ea_flat, dst_h, z_deg)

    dis = pl.pallas_call(
        _r_body,
        out_shape=jax.ShapeDtypeStruct((N, 16), f32),
    )(degp)

    phase_b = pl.kernel(
        _b_body,
        out_type=jax.ShapeDtypeStruct((8, N, 4, 16), f32),
        mesh=mesh,
        compiler_params=pltpu.CompilerParams(needs_layout_passes=False, use_tc_tiling_on_sc=False),
        scratch_types=[
            pltpu.VMEM_SHARED((N, 16), f32),
            pltpu.VMEM_SHARED((N, 16), f32),
            pltpu.VMEM_SHARED((N, 4, 16), f32),
            pltpu.VMEM((B_BLK,), jnp.int32),
            pltpu.VMEM((B_BLK,), jnp.int32),
            pltpu.VMEM((B_BLK * 4,), f32),
            pltpu.VMEM((B_BLK * 4,), f32),
            pltpu.VMEM((B_BLK, 16), f32),
            pltpu.VMEM((B_BLK, 16), f32),
            pltpu.VMEM((B_BLK, 4, 16), f32),
        ],
    )
    xt = x.reshape(N, 8, 16).transpose(1, 0, 2)
    z_s = jnp.zeros((N, 4, 16), f32)
    s_acc = phase_b(src_h, dst_h, w4.reshape(E * 4), dis, xt, z_s)

    bn = 1000
    out = pl.pallas_call(
        _c_body,
        grid=(N // bn,),
        in_specs=[
            pl.BlockSpec((bn, D), lambda j: (j, 0)),
            pl.BlockSpec((8, bn, 4, 16), lambda j: (0, j, 0, 0)),
            pl.BlockSpec((NC, bn, 16), lambda j: (0, j, 0)),
            pl.BlockSpec((4, D, D), lambda j: (0, 0, 0)),
            pl.BlockSpec((4, D), lambda j: (0, 0)),
            pl.BlockSpec((4, D, D), lambda j: (0, 0, 0)),
        ],
        out_specs=pl.BlockSpec((bn, D), lambda j: (j, 0)),
        out_shape=jax.ShapeDtypeStruct((N, D), f32),
    )(x, s_acc, degp, gcn_W, gcn_b, Ws)
    return out


# speedup vs baseline: 9.0666x; 1.0252x over previous
import functools

import jax
import jax.numpy as jnp
from jax import lax
from jax.experimental import pallas as pl
from jax.experimental.pallas import tpu as pltpu
from jax.experimental.pallas import tpu_sc as plsc

N = 10000
E = 320000
D = 128
NT = 16
NC = 2
CT = 10
CROWS = N // CT

A_BLK = 1000
A_EPW = E // (NC * NT)
B_BLK = 400
B_EPW = E // NT


def _a_body(ea_ref, dst_ref, zdeg_ref, degp_ref, w4_ref, deg_sp, ea_v, dst_v,
            w4b16_v, w4b4_v):
    c = lax.axis_index("c")
    s = lax.axis_index("s")
    w = c * NT + s
    crows = pl.ds(s * CROWS, CROWS)

    @pl.when(s < CT)
    def _():
        pltpu.sync_copy(zdeg_ref.at[crows], deg_sp.at[crows])

    plsc.subcore_barrier()

    l = lax.iota(jnp.int32, 16)
    gpat = (l >> 2) * 16 + (l & 3) + 2
    rpat = l >> 2
    cpat = l & 3

    pltpu.sync_copy(zdeg_ref.at[pl.ds(0, A_BLK)], w4b16_v)

    def blk(b, carry):
        base = pl.multiple_of(w * A_EPW + b * A_BLK, 8)
        pltpu.sync_copy(ea_ref.at[pl.ds(base * 16, A_BLK * 16)], ea_v)
        pltpu.sync_copy(dst_ref.at[pl.ds(base, A_BLK)], dst_v)

        def ext(j, carry2):
            w16 = plsc.load_gather(ea_v, [gpat + j * 64])
            plsc.store_scatter(w4b16_v, [rpat + j * 4, cpat], w16)
            plsc.store_scatter(w4b4_v, [rpat + j * 4, cpat], w16)
            return carry2

        lax.fori_loop(0, A_BLK // 4, ext, 0)
        pltpu.sync_copy(w4b4_v, w4_ref.at[pl.ds(base, A_BLK)])
        pltpu.sync_copy(w4b16_v, deg_sp.at[dst_v], add=True)
        return carry

    lax.fori_loop(0, A_EPW // A_BLK, blk, 0)
    plsc.subcore_barrier()

    @pl.when(s < CT)
    def _():
        pltpu.sync_copy(deg_sp.at[crows], degp_ref.at[c, crows])


def _r_body(degp_ref, dis_ref):
    deg = degp_ref[0, :, 0:4] + degp_ref[1, :, 0:4] + 1.0
    dis_ref[...] = jnp.concatenate(
        [lax.rsqrt(deg), jnp.zeros((deg.shape[0], 12), jnp.float32)], axis=-1)


def _b_body(src_ref, dst_ref, w4_ref, dis_ref, xt_ref, zs_ref, s_ref,
            dis_sp, x_sp, s_sp, src_v, dst_v, w_v, a_v, disg_v, xg_v, msg_v):
    c = lax.axis_index("c")
    s = lax.axis_index("s")
    crows = pl.ds(s * CROWS, CROWS)

    @pl.when(s < CT)
    def _():
        pltpu.sync_copy(dis_ref.at[crows], dis_sp.at[crows])

    l = lax.iota(jnp.int32, 16)
    rpat = l >> 2
    cpat = l & 3

    def quarter(qi, carry):
        q = c * 4 + qi

        @pl.when(s < CT)
        def _():
            pltpu.sync_copy(xt_ref.at[q, crows], x_sp.at[crows])
            pltpu.sync_copy(zs_ref.at[crows], s_sp.at[crows])

        plsc.subcore_barrier()

        def blk(b, carry2):
            base = pl.multiple_of(s * B_EPW + b * B_BLK, 8)
            pltpu.sync_copy(src_ref.at[pl.ds(base, B_BLK)], src_v)
            pltpu.sync_copy(dst_ref.at[pl.ds(base, B_BLK)], dst_v)
            pltpu.sync_copy(w4_ref.at[pl.ds(base * 4, B_BLK * 4)], w_v)
            pltpu.sync_copy(dis_sp.at[src_v], disg_v)
            pltpu.sync_copy(x_sp.at[src_v], xg_v)

            def agrp(j, carry3):
                a16 = (plsc.load_gather(disg_v, [rpat + j * 4, cpat])
                       * w_v[pl.ds(j * 16, 16)])
                a_v[pl.ds(j * 16, 16)] = a16
                return carry3

            lax.fori_loop(0, B_BLK * 4 // 16, agrp, 0)

            def edge(eu, carry3):
                for ee in range(4):
                    e = eu * 4 + ee
                    xv = xg_v[e, pl.ds(0, 16)]
                    for i in range(4):
                        ab = plsc.load_gather(
                            a_v, [jnp.broadcast_to(e * 4 + i, (16,))])
                        msg_v[e, i, pl.ds(0, 16)] = ab * xv
                return carry3

            lax.fori_loop(0, B_BLK // 4, edge, 0)
            pltpu.sync_copy(msg_v, s_sp.at[dst_v], add=True)
            return carry2

        lax.fori_loop(0, B_EPW // B_BLK, blk, 0)
        plsc.subcore_barrier()

        @pl.when(s < CT)
        def _():
            pltpu.sync_copy(s_sp.at[crows], s_ref.at[q, crows])

        return carry

    lax.fori_loop(0, 4, quarter, 0)


def _c_body(x_ref, s_ref, degp_ref, w_ref, b_ref, ws_ref, o_ref):
    deg = degp_ref[0, :, 0:4] + degp_ref[1, :, 0:4] + 1.0
    dis = lax.rsqrt(deg)
    dinv = 1.0 / deg
    xb = x_ref[...]
    acc = jnp.zeros(o_ref.shape, jnp.float32)
    for i in range(4):
        si = jnp.concatenate([s_ref[q, :, i, :] for q in range(8)], axis=-1)
        g = dis[:, i:i + 1] * si + dinv[:, i:i + 1] * xb
        agg = jnp.maximum(jnp.dot(g, w_ref[i], preferred_element_type=jnp.float32)
                          + b_ref[i], 0.0)
        acc = acc + jnp.dot(agg, ws_ref[i], preferred_element_type=jnp.float32)
    o_ref[...] = acc


@jax.jit
def kernel(x, edge_index, edge_attr, gcn_W, gcn_b, Ws):
    f32 = jnp.float32
    mesh = plsc.VectorSubcoreMesh(core_axis_name="c", subcore_axis_name="s")

    phase_a = pl.kernel(
        _a_body,
        out_type=(jax.ShapeDtypeStruct((NC, N, 16), f32),
                  jax.ShapeDtypeStruct((E, 4), f32)),
        mesh=mesh,
        compiler_params=pltpu.CompilerParams(needs_layout_passes=False, use_tc_tiling_on_sc=False),
        scratch_types=[
            pltpu.VMEM_SHARED((N, 16), f32),
            pltpu.VMEM((A_BLK * 16,), f32),
            pltpu.VMEM((A_BLK,), jnp.int32),
            pltpu.VMEM((A_BLK, 16), f32),
            pltpu.VMEM((A_BLK, 4), f32),
        ],
    )
    ea_flat = edge_attr.reshape(E * 16)
    src_h = edge_index[0]
    dst_h = edge_index[1]
    z_deg = jnp.zeros((N, 16), f32)
    degp, w4 = phase_a(ea_flat, dst_h, z_deg)

    dis = pl.pallas_call(
        _r_body,
        out_shape=jax.ShapeDtypeStruct((N, 16), f32),
    )(degp)

    phase_b = pl.kernel(
        _b_body,
        out_type=jax.ShapeDtypeStruct((8, N, 4, 16), f32),
        mesh=mesh,
        compiler_params=pltpu.CompilerParams(needs_layout_passes=False, use_tc_tiling_on_sc=False),
        scratch_types=[
            pltpu.VMEM_SHARED((N, 16), f32),
            pltpu.VMEM_SHARED((N, 16), f32),
            pltpu.VMEM_SHARED((N, 4, 16), f32),
            pltpu.VMEM((B_BLK,), jnp.int32),
            pltpu.VMEM((B_BLK,), jnp.int32),
            pltpu.VMEM((B_BLK * 4,), f32),
            pltpu.VMEM((B_BLK * 4,), f32),
            pltpu.VMEM((B_BLK, 16), f32),
            pltpu.VMEM((B_BLK, 16), f32),
            pltpu.VMEM((B_BLK, 4, 16), f32),
        ],
    )
    xt = x.reshape(N, 8, 16).transpose(1, 0, 2)
    z_s = jnp.zeros((N, 4, 16), f32)
    s_acc = phase_b(src_h, dst_h, w4.reshape(E * 4), dis, xt, z_s)

    bn = 1000
    out = pl.pallas_call(
        _c_body,
        grid=(N // bn,),
        in_specs=[
            pl.BlockSpec((bn, D), lambda j: (j, 0)),
            pl.BlockSpec((8, bn, 4, 16), lambda j: (0, j, 0, 0)),
            pl.BlockSpec((NC, bn, 16), lambda j: (0, j, 0)),
            pl.BlockSpec((4, D, D), lambda j: (0, 0, 0)),
            pl.BlockSpec((4, D), lambda j: (0, 0)),
            pl.BlockSpec((4, D, D), lambda j: (0, 0, 0)),
        ],
        out_specs=pl.BlockSpec((bn, D), lambda j: (j, 0)),
        out_shape=jax.ShapeDtypeStruct((N, D), f32),
    )(x, s_acc, degp, gcn_W, gcn_b, Ws)
    return out
